# Initial kernel scaffold; baseline (speedup 1.0000x reference)
import functools

import jax
import jax.numpy as jnp
from jax import lax
from jax.experimental import pallas as pl
from jax.experimental.pallas import tpu as pltpu
from jax.experimental.pallas import tpu_sc as plsc

F32 = jnp.float32
_NC = 2
_NS = 16
_L = 16
_C = 80


def _bcast_lane(v, h):
    idx = jnp.full((_L, 1), h, jnp.int32)
    dn = lax.GatherDimensionNumbers(
        offset_dims=(), collapsed_slice_dims=(0,), start_index_map=(0,))
    return lax.gather(v, idx, dn, slice_sizes=(1,),
                      mode=lax.GatherScatterMode.PROMISE_IN_BOUNDS)


def _edge_pass(tsrc, tdst, src, dst, zeros_hbm):
    n = tsrc.shape[0]
    e = src.shape[0]
    nw = _NC * _NS
    epw = e // nw
    nchunks = epw // _C
    rpt = n // _NS
    mesh = plsc.VectorSubcoreMesh(core_axis_name="c", subcore_axis_name="s",
                                  num_cores=_NC, num_subcores=_NS)

    @functools.partial(
        pl.kernel,
        out_type=jax.ShapeDtypeStruct((_NC, n, 144), F32),
        mesh=mesh,
        scratch_types=[
            pltpu.VMEM((_C,), jnp.int32),
            pltpu.VMEM((_C,), jnp.int32),
            pltpu.VMEM((_C, 144), F32),
            pltpu.VMEM((_C, 16), F32),
            pltpu.VMEM_SHARED((n, 144), F32),
            pltpu.SemaphoreType.DMA,
            pltpu.SemaphoreType.DMA,
        ],
    )
    def k(tsrc_hbm, tdst_hbm, src_hbm, dst_hbm, z_hbm, out_hbm,
          sidx, didx, srows, drows, acc, sem1, sem2):
        c = lax.axis_index("c")
        s = lax.axis_index("s")
        wid = c * _NS + s

        pltpu.sync_copy(z_hbm.at[pl.ds(s * rpt, rpt)],
                        acc.at[pl.ds(s * rpt, rpt)])
        plsc.subcore_barrier()

        @pl.loop(0, nchunks)
        def _(i):
            base = wid * epw + i * _C
            pltpu.sync_copy(src_hbm.at[pl.ds(base, _C)], sidx)
            pltpu.sync_copy(dst_hbm.at[pl.ds(base, _C)], didx)
            cp1 = pltpu.async_copy(tsrc_hbm.at[sidx], srows, sem1)
            cp2 = pltpu.async_copy(tdst_hbm.at[didx], drows, sem2)
            cp1.wait()
            cp2.wait()

            @pl.loop(0, _C)
            def _(ei):
                a = srows[ei, pl.ds(128, _L)] + drows[ei, pl.ds(0, _L)]
                a = jnp.maximum(a, a * 0.2)
                ex = jnp.exp(a)
                srows[ei, pl.ds(128, _L)] = ex
                for h in range(8):
                    bh = _bcast_lane(ex, h)
                    srows[ei, pl.ds(h * _L, _L)] = (
                        srows[ei, pl.ds(h * _L, _L)] * bh)

            pltpu.sync_copy(srows, acc.at[didx], add=True)

        plsc.subcore_barrier()
        pltpu.sync_copy(acc.at[pl.ds(s * rpt, rpt)],
                        out_hbm.at[c, pl.ds(s * rpt, rpt)])

    return k(tsrc, tdst, src, dst, zeros_hbm)


def _prep_tc(x, w_ext, w_dst):
    n = x.shape[0]

    def body(x_ref, we_ref, wd_ref, ts_ref, td_ref):
        xx = x_ref[...]
        ts_ref[...] = jnp.dot(xx, we_ref[...], preferred_element_type=F32)
        td_ref[...] = jnp.dot(xx, wd_ref[...], preferred_element_type=F32)

    return pl.pallas_call(
        body,
        out_shape=(jax.ShapeDtypeStruct((n, 144), F32),
                   jax.ShapeDtypeStruct((n, 16), F32)),
    )(x, w_ext, w_dst)


def _mid_tc(acc, b1, w_ext, w_dst, p8):
    n = acc.shape[1]

    def body(acc_ref, b1_ref, we_ref, wd_ref, p8_ref, ts_ref, td_ref):
        a = acc_ref[0] + acc_ref[1]
        num = a[:, :128]
        den = a[:, 128:136]
        r = 1.0 / (den + 1e-16)
        rex = jnp.dot(r, p8_ref[...], preferred_element_type=F32)
        hpre = num * rex + b1_ref[...]
        hh = jnp.where(hpre > 0, hpre, jnp.exp(hpre) - 1.0)
        ts_ref[...] = jnp.dot(hh, we_ref[...], preferred_element_type=F32)
        td_ref[...] = jnp.dot(hh, wd_ref[...], preferred_element_type=F32)

    return pl.pallas_call(
        body,
        out_shape=(jax.ShapeDtypeStruct((n, 144), F32),
                   jax.ShapeDtypeStruct((n, 16), F32)),
    )(acc, b1, w_ext, w_dst, p8)


def _final_tc(acc, b2, p0):
    n = acc.shape[1]

    def body(acc_ref, b2_ref, p0_ref, o_ref):
        a = acc_ref[0] + acc_ref[1]
        num = a[:, :128]
        den = a[:, 128:136]
        r = 1.0 / (den + 1e-16)
        rex = jnp.dot(r, p0_ref[...], preferred_element_type=F32)
        o = num * rex + b2_ref[...]
        m = jnp.max(o, axis=1, keepdims=True)
        z = o - m
        lse = jnp.log(jnp.sum(jnp.exp(z), axis=1, keepdims=True))
        o_ref[...] = z - lse

    return pl.pallas_call(
        body,
        out_shape=jax.ShapeDtypeStruct((n, 128), F32),
    )(acc, b2, p0)


def kernel(x, edge_index, W1, att_src1, att_dst1, b1, W2, att_src2,
           att_dst2, b2):
    n = x.shape[0]
    heads, hid = att_src1.shape
    src = edge_index[0]
    dst = edge_index[1]

    eye = jnp.eye(heads, dtype=F32)
    a_s = (eye[:, None, :] * att_src1[:, :, None]).reshape(heads * hid, heads)
    a_d = (eye[:, None, :] * att_dst1[:, :, None]).reshape(heads * hid, heads)
    z8 = jnp.zeros((heads * hid, 8), F32)
    w1_ext = jnp.concatenate([W1, W1 @ a_s, z8], axis=1)
    w1_dst = jnp.concatenate([W1 @ a_d, z8], axis=1)
    w2_ext = jnp.concatenate(
        [W2, jnp.tile((W2 @ att_src2[0])[:, None], (1, 16))], axis=1)
    w2_dst = jnp.tile((W2 @ att_dst2[0])[:, None], (1, 16))
    p8 = (jnp.arange(128)[None, :] // hid
          == jnp.arange(heads)[:, None]).astype(F32)
    p0 = jnp.concatenate([jnp.ones((1, 128), F32),
                          jnp.zeros((7, 128), F32)], axis=0)
    zeros_hbm = jnp.zeros((n, 144), F32)

    tsrc1, tdst1 = _prep_tc(x, w1_ext, w1_dst)
    acc1 = _edge_pass(tsrc1, tdst1, src, dst, zeros_hbm)
    tsrc2, tdst2 = _mid_tc(acc1, b1.reshape(1, -1), w2_ext, w2_dst, p8)
    acc2 = _edge_pass(tsrc2, tdst2, src, dst, zeros_hbm)
    return _final_tc(acc2, b2.reshape(1, -1), p0)


# speedup vs baseline: 42.9871x; 42.9871x over previous
import functools

import jax
import jax.numpy as jnp
from jax import lax
from jax.experimental import pallas as pl
from jax.experimental.pallas import tpu as pltpu
from jax.experimental.pallas import tpu_sc as plsc

F32 = jnp.float32
_NC = 2
_NS = 16
_L = 16
_C = 80


def _bcast_lane(v, h):
    idx = jnp.full((_L, 1), h, jnp.int32)
    dn = lax.GatherDimensionNumbers(
        offset_dims=(), collapsed_slice_dims=(0,), start_index_map=(0,))
    return lax.gather(v, idx, dn, slice_sizes=(1,),
                      mode=lax.GatherScatterMode.PROMISE_IN_BOUNDS)


def _edge_pass(tsrc, tdst, src, dst, zeros_hbm):
    n = tsrc.shape[0]
    e = src.shape[0]
    nw = _NC * _NS
    epw = e // nw
    nchunks = epw // _C
    rpt = (n // _NS) // 8 * 8
    rem = n - rpt * _NS
    mesh = plsc.VectorSubcoreMesh(core_axis_name="c", subcore_axis_name="s",
                                  num_cores=_NC, num_subcores=_NS)

    @functools.partial(
        pl.kernel,
        out_type=jax.ShapeDtypeStruct((_NC, n, 144), F32),
        mesh=mesh,
        compiler_params=pltpu.CompilerParams(use_tc_tiling_on_sc=False),
        scratch_types=[
            pltpu.VMEM((_C,), jnp.int32),
            pltpu.VMEM((_C,), jnp.int32),
            pltpu.VMEM((_C, 144), F32),
            pltpu.VMEM((_C, 16), F32),
            pltpu.VMEM_SHARED((n, 144), F32),
            pltpu.SemaphoreType.DMA,
            pltpu.SemaphoreType.DMA,
        ],
    )
    def k(tsrc_hbm, tdst_hbm, src_hbm, dst_hbm, z_hbm, out_hbm,
          sidx, didx, srows, drows, acc, sem1, sem2):
        c = lax.axis_index("c")
        s = lax.axis_index("s")
        wid = c * _NS + s

        pltpu.sync_copy(z_hbm.at[pl.ds(s * rpt, rpt)],
                        acc.at[pl.ds(s * rpt, rpt)])

        @pl.when(s == 0)
        def _():
            pltpu.sync_copy(z_hbm.at[pl.ds(rpt * _NS, rem)],
                            acc.at[pl.ds(rpt * _NS, rem)])

        plsc.subcore_barrier()

        @pl.loop(0, nchunks)
        def _(i):
            base = wid * epw + i * _C
            pltpu.sync_copy(src_hbm.at[pl.ds(base, _C)], sidx)
            pltpu.sync_copy(dst_hbm.at[pl.ds(base, _C)], didx)
            cp1 = pltpu.async_copy(tsrc_hbm.at[sidx], srows, sem1)
            cp2 = pltpu.async_copy(tdst_hbm.at[didx], drows, sem2)
            cp1.wait()
            cp2.wait()

            @pl.loop(0, _C)
            def _(ei):
                a = srows[ei, pl.ds(128, _L)] + drows[ei, pl.ds(0, _L)]
                a = jnp.maximum(a, a * 0.2)
                ex = jnp.exp(a)
                srows[ei, pl.ds(128, _L)] = ex
                for h in range(8):
                    bh = _bcast_lane(ex, h)
                    srows[ei, pl.ds(h * _L, _L)] = (
                        srows[ei, pl.ds(h * _L, _L)] * bh)

            pltpu.sync_copy(srows, acc.at[didx], add=True)

        plsc.subcore_barrier()
        pltpu.sync_copy(acc.at[pl.ds(s * rpt, rpt)],
                        out_hbm.at[c, pl.ds(s * rpt, rpt)])

        @pl.when(s == 0)
        def _():
            pltpu.sync_copy(acc.at[pl.ds(rpt * _NS, rem)],
                            out_hbm.at[c, pl.ds(rpt * _NS, rem)])

    return k(tsrc, tdst, src, dst, zeros_hbm)


def _prep_tc(x, w_ext, w_dst):
    n = x.shape[0]

    def body(x_ref, we_ref, wd_ref, ts_ref, td_ref):
        xx = x_ref[...]
        ts_ref[...] = jnp.dot(xx, we_ref[...], preferred_element_type=F32)
        td_ref[...] = jnp.dot(xx, wd_ref[...], preferred_element_type=F32)

    return pl.pallas_call(
        body,
        out_shape=(jax.ShapeDtypeStruct((n, 144), F32),
                   jax.ShapeDtypeStruct((n, 16), F32)),
    )(x, w_ext, w_dst)


def _mid_tc(acc, b1, w_ext, w_dst, p8):
    n = acc.shape[1]

    def body(acc_ref, b1_ref, we_ref, wd_ref, p8_ref, ts_ref, td_ref):
        a = acc_ref[0] + acc_ref[1]
        num = a[:, :128]
        den = a[:, 128:136]
        r = 1.0 / (den + 1e-16)
        rex = jnp.dot(r, p8_ref[...], preferred_element_type=F32)
        hpre = num * rex + b1_ref[...]
        hh = jnp.where(hpre > 0, hpre, jnp.exp(hpre) - 1.0)
        ts_ref[...] = jnp.dot(hh, we_ref[...], preferred_element_type=F32)
        td_ref[...] = jnp.dot(hh, wd_ref[...], preferred_element_type=F32)

    return pl.pallas_call(
        body,
        out_shape=(jax.ShapeDtypeStruct((n, 144), F32),
                   jax.ShapeDtypeStruct((n, 16), F32)),
    )(acc, b1, w_ext, w_dst, p8)


def _final_tc(acc, b2, p0):
    n = acc.shape[1]

    def body(acc_ref, b2_ref, p0_ref, o_ref):
        a = acc_ref[0] + acc_ref[1]
        num = a[:, :128]
        den = a[:, 128:136]
        r = 1.0 / (den + 1e-16)
        rex = jnp.dot(r, p0_ref[...], preferred_element_type=F32)
        o = num * rex + b2_ref[...]
        m = jnp.max(o, axis=1, keepdims=True)
        z = o - m
        lse = jnp.log(jnp.sum(jnp.exp(z), axis=1, keepdims=True))
        o_ref[...] = z - lse

    return pl.pallas_call(
        body,
        out_shape=jax.ShapeDtypeStruct((n, 128), F32),
    )(acc, b2, p0)


def kernel(x, edge_index, W1, att_src1, att_dst1, b1, W2, att_src2,
           att_dst2, b2):
    n = x.shape[0]
    heads, hid = att_src1.shape
    src = edge_index[0]
    dst = edge_index[1]

    eye = jnp.eye(heads, dtype=F32)
    a_s = (eye[:, None, :] * att_src1[:, :, None]).reshape(heads * hid, heads)
    a_d = (eye[:, None, :] * att_dst1[:, :, None]).reshape(heads * hid, heads)
    z8 = jnp.zeros((heads * hid, 8), F32)
    w1_ext = jnp.concatenate([W1, W1 @ a_s, z8], axis=1)
    w1_dst = jnp.concatenate([W1 @ a_d, z8], axis=1)
    w2_ext = jnp.concatenate(
        [W2, jnp.tile((W2 @ att_src2[0])[:, None], (1, 16))], axis=1)
    w2_dst = jnp.tile((W2 @ att_dst2[0])[:, None], (1, 16))
    p8 = (jnp.arange(128)[None, :] // hid
          == jnp.arange(heads)[:, None]).astype(F32)
    p0 = jnp.concatenate([jnp.ones((1, 128), F32),
                          jnp.zeros((7, 128), F32)], axis=0)
    zeros_hbm = jnp.zeros((n, 144), F32)

    tsrc1, tdst1 = _prep_tc(x, w1_ext, w1_dst)
    acc1 = _edge_pass(tsrc1, tdst1, src, dst, zeros_hbm)
    tsrc2, tdst2 = _mid_tc(acc1, b1.reshape(1, -1), w2_ext, w2_dst, p8)
    acc2 = _edge_pass(tsrc2, tdst2, src, dst, zeros_hbm)
    return _final_tc(acc2, b2.reshape(1, -1), p0)


# speedup vs baseline: 72.8484x; 1.6947x over previous
import functools

import jax
import jax.numpy as jnp
from jax import lax
from jax.experimental import pallas as pl
from jax.experimental.pallas import tpu as pltpu
from jax.experimental.pallas import tpu_sc as plsc

F32 = jnp.float32
_NC = 2
_NS = 16
_L = 16
_C = 40


def _bcast_lane(v, h):
    idx = jnp.full((_L, 1), h, jnp.int32)
    dn = lax.GatherDimensionNumbers(
        offset_dims=(), collapsed_slice_dims=(0,), start_index_map=(0,))
    return lax.gather(v, idx, dn, slice_sizes=(1,),
                      mode=lax.GatherScatterMode.PROMISE_IN_BOUNDS)


def _edge_pass(tsrc, tdst, eix, zeros_hbm):
    n = tsrc.shape[0]
    nchunks_tot = eix.shape[1]
    nw = _NC * _NS
    cpw = nchunks_tot // nw
    rpt = (n // _NS) // 8 * 8
    rem = n - rpt * _NS
    mesh = plsc.VectorSubcoreMesh(core_axis_name="c", subcore_axis_name="s",
                                  num_cores=_NC, num_subcores=_NS)

    @functools.partial(
        pl.kernel,
        out_type=jax.ShapeDtypeStruct((_NC, n, 144), F32),
        mesh=mesh,
        compiler_params=pltpu.CompilerParams(use_tc_tiling_on_sc=False),
        scratch_types=[
            pltpu.VMEM((2, cpw, _C), jnp.int32),
            pltpu.VMEM((_C, 144), F32),
            pltpu.VMEM((_C, 144), F32),
            pltpu.VMEM((_C, 16), F32),
            pltpu.VMEM((_C, 16), F32),
            pltpu.VMEM((_C, 144), F32),
            pltpu.VMEM_SHARED((n, 144), F32),
            pltpu.SemaphoreType.DMA,
            pltpu.SemaphoreType.DMA,
            pltpu.SemaphoreType.DMA,
            pltpu.SemaphoreType.DMA,
            pltpu.SemaphoreType.DMA,
        ],
    )
    def k(tsrc_hbm, tdst_hbm, eix_hbm, z_hbm, out_hbm,
          idxb, g0, g1, d0, d1, ob, acc,
          sgs0, sgs1, sgd0, sgd1, ssc):
        c = lax.axis_index("c")
        s = lax.axis_index("s")
        wid = c * _NS + s
        gslot = (g0, g1)
        dslot = (d0, d1)
        gsem = (sgs0, sgs1)
        dsem = (sgd0, sgd1)

        cpi = pltpu.make_async_copy(
            eix_hbm.at[:, pl.ds(wid * cpw, cpw), :], idxb, sgs0)
        cpi.start()

        pltpu.sync_copy(z_hbm.at[pl.ds(s * rpt, rpt)],
                        acc.at[pl.ds(s * rpt, rpt)])

        @pl.when(s == 0)
        def _():
            pltpu.sync_copy(z_hbm.at[pl.ds(rpt * _NS, rem)],
                            acc.at[pl.ds(rpt * _NS, rem)])

        cpi.wait()
        plsc.subcore_barrier()

        def gather_desc(chunk, slot):
            cg = pltpu.make_async_copy(
                tsrc_hbm.at[idxb.at[0, chunk]], gslot[slot], gsem[slot])
            cd = pltpu.make_async_copy(
                tdst_hbm.at[idxb.at[1, chunk]], dslot[slot], dsem[slot])
            return cg, cd

        def issue_gather(chunk, slot):
            cg, cd = gather_desc(chunk, slot)
            cg.start()
            cd.start()

        def wait_gather(chunk, slot):
            cg, cd = gather_desc(chunk, slot)
            cg.wait()
            cd.wait()

        def compute(slot):
            g, d = gslot[slot], dslot[slot]

            @pl.loop(0, _C)
            def _(ei):
                a = g[ei, pl.ds(128, _L)] + d[ei, pl.ds(0, _L)]
                a = jnp.maximum(a, a * 0.2)
                ex = jnp.exp(a)
                ob[ei, pl.ds(128, _L)] = ex
                for h in range(8):
                    bh = _bcast_lane(ex, h)
                    ob[ei, pl.ds(h * _L, _L)] = g[ei, pl.ds(h * _L, _L)] * bh

        def scatter_desc(chunk):
            return pltpu.make_async_copy(
                ob, acc.at[idxb.at[1, chunk]], ssc)

        issue_gather(0, 0)
        issue_gather(1, 1)

        wait_gather(0, 0)
        compute(0)
        scatter_desc(0).start(add=True)
        issue_gather(2, 0)

        @pl.loop(1, cpw)
        def _(chunk):
            slot = lax.rem(chunk, 2)

            @pl.when(slot == 0)
            def _():
                wait_gather(chunk, 0)

            @pl.when(slot == 1)
            def _():
                wait_gather(chunk, 1)

            scatter_desc(chunk - 1).wait()

            @pl.when(slot == 0)
            def _():
                compute(0)

            @pl.when(slot == 1)
            def _():
                compute(1)

            scatter_desc(chunk).start(add=True)

            @pl.when((slot == 0) & (chunk + 2 < cpw))
            def _():
                issue_gather(chunk + 2, 0)

            @pl.when((slot == 1) & (chunk + 2 < cpw))
            def _():
                issue_gather(chunk + 2, 1)

        scatter_desc(cpw - 1).wait()

        plsc.subcore_barrier()
        pltpu.sync_copy(acc.at[pl.ds(s * rpt, rpt)],
                        out_hbm.at[c, pl.ds(s * rpt, rpt)])

        @pl.when(s == 0)
        def _():
            pltpu.sync_copy(acc.at[pl.ds(rpt * _NS, rem)],
                            out_hbm.at[c, pl.ds(rpt * _NS, rem)])

    return k(tsrc, tdst, eix, zeros_hbm)


def _prep_tc(x, w_ext, w_dst):
    n = x.shape[0]

    def body(x_ref, we_ref, wd_ref, ts_ref, td_ref):
        xx = x_ref[...]
        ts_ref[...] = jnp.dot(xx, we_ref[...], preferred_element_type=F32)
        td_ref[...] = jnp.dot(xx, wd_ref[...], preferred_element_type=F32)

    return pl.pallas_call(
        body,
        out_shape=(jax.ShapeDtypeStruct((n, 144), F32),
                   jax.ShapeDtypeStruct((n, 16), F32)),
    )(x, w_ext, w_dst)


def _mid_tc(acc, b1, w_ext, w_dst, p8):
    n = acc.shape[1]

    def body(acc_ref, b1_ref, we_ref, wd_ref, p8_ref, ts_ref, td_ref):
        a = acc_ref[0] + acc_ref[1]
        num = a[:, :128]
        den = a[:, 128:136]
        r = 1.0 / (den + 1e-16)
        rex = jnp.dot(r, p8_ref[...], preferred_element_type=F32)
        hpre = num * rex + b1_ref[...]
        hh = jnp.where(hpre > 0, hpre, jnp.exp(hpre) - 1.0)
        ts_ref[...] = jnp.dot(hh, we_ref[...], preferred_element_type=F32)
        td_ref[...] = jnp.dot(hh, wd_ref[...], preferred_element_type=F32)

    return pl.pallas_call(
        body,
        out_shape=(jax.ShapeDtypeStruct((n, 144), F32),
                   jax.ShapeDtypeStruct((n, 16), F32)),
    )(acc, b1, w_ext, w_dst, p8)


def _final_tc(acc, b2, p0):
    n = acc.shape[1]

    def body(acc_ref, b2_ref, p0_ref, o_ref):
        a = acc_ref[0] + acc_ref[1]
        num = a[:, :128]
        den = a[:, 128:136]
        r = 1.0 / (den + 1e-16)
        rex = jnp.dot(r, p0_ref[...], preferred_element_type=F32)
        o = num * rex + b2_ref[...]
        m = jnp.max(o, axis=1, keepdims=True)
        z = o - m
        lse = jnp.log(jnp.sum(jnp.exp(z), axis=1, keepdims=True))
        o_ref[...] = z - lse

    return pl.pallas_call(
        body,
        out_shape=jax.ShapeDtypeStruct((n, 128), F32),
    )(acc, b2, p0)


def kernel(x, edge_index, W1, att_src1, att_dst1, b1, W2, att_src2,
           att_dst2, b2):
    n = x.shape[0]
    heads, hid = att_src1.shape
    eix = edge_index.reshape(2, -1, _C)

    eye = jnp.eye(heads, dtype=F32)
    a_s = (eye[:, None, :] * att_src1[:, :, None]).reshape(heads * hid, heads)
    a_d = (eye[:, None, :] * att_dst1[:, :, None]).reshape(heads * hid, heads)
    z8 = jnp.zeros((heads * hid, 8), F32)
    w1_ext = jnp.concatenate([W1, W1 @ a_s, z8], axis=1)
    w1_dst = jnp.concatenate([W1 @ a_d, z8], axis=1)
    w2_ext = jnp.concatenate(
        [W2, jnp.tile((W2 @ att_src2[0])[:, None], (1, 16))], axis=1)
    w2_dst = jnp.tile((W2 @ att_dst2[0])[:, None], (1, 16))
    p8 = (jnp.arange(128)[None, :] // hid
          == jnp.arange(heads)[:, None]).astype(F32)
    p0 = jnp.concatenate([jnp.ones((1, 128), F32),
                          jnp.zeros((7, 128), F32)], axis=0)
    zeros_hbm = jnp.zeros((n, 144), F32)

    tsrc1, tdst1 = _prep_tc(x, w1_ext, w1_dst)
    acc1 = _edge_pass(tsrc1, tdst1, eix, zeros_hbm)
    tsrc2, tdst2 = _mid_tc(acc1, b1.reshape(1, -1), w2_ext, w2_dst, p8)
    acc2 = _edge_pass(tsrc2, tdst2, eix, zeros_hbm)
    return _final_tc(acc2, b2.reshape(1, -1), p0)


# speedup vs baseline: 106.9203x; 1.4677x over previous
import functools

import jax
import jax.numpy as jnp
from jax import lax
from jax.experimental import pallas as pl
from jax.experimental.pallas import tpu as pltpu
from jax.experimental.pallas import tpu_sc as plsc

F32 = jnp.float32
_NC = 2
_NS = 16
_L = 16
_C = 40


def _bcast_lane(v, h):
    idx = jnp.full((_L, 1), h, jnp.int32)
    dn = lax.GatherDimensionNumbers(
        offset_dims=(), collapsed_slice_dims=(0,), start_index_map=(0,))
    return lax.gather(v, idx, dn, slice_sizes=(1,),
                      mode=lax.GatherScatterMode.PROMISE_IN_BOUNDS)


def _edge_pass(tsrc, tdst, eix, zeros_hbm):
    n = tsrc.shape[0]
    nchunks_tot = eix.shape[1]
    nw = _NC * _NS
    cpw = nchunks_tot // nw
    rpt = (n // _NS) // 8 * 8
    rem = n - rpt * _NS
    mesh = plsc.VectorSubcoreMesh(core_axis_name="c", subcore_axis_name="s",
                                  num_cores=_NC, num_subcores=_NS)

    @functools.partial(
        pl.kernel,
        out_type=jax.ShapeDtypeStruct((_NC, n, 144), F32),
        mesh=mesh,
        compiler_params=pltpu.CompilerParams(use_tc_tiling_on_sc=False),
        scratch_types=[
            pltpu.VMEM((2, cpw, _C), jnp.int32),
            pltpu.VMEM((_C, 144), F32),
            pltpu.VMEM((_C, 144), F32),
            pltpu.VMEM((_C, 16), F32),
            pltpu.VMEM((_C, 16), F32),
            pltpu.VMEM((_C, 144), F32),
            pltpu.VMEM_SHARED((n, 144), F32),
            pltpu.SemaphoreType.DMA,
            pltpu.SemaphoreType.DMA,
            pltpu.SemaphoreType.DMA,
            pltpu.SemaphoreType.DMA,
            pltpu.SemaphoreType.DMA,
        ],
    )
    def k(tsrc_hbm, tdst_hbm, eix_hbm, z_hbm, out_hbm,
          idxb, g0, g1, d0, d1, ob, acc,
          sgs0, sgs1, sgd0, sgd1, ssc):
        c = lax.axis_index("c")
        s = lax.axis_index("s")
        wid = c * _NS + s
        gslot = (g0, g1)
        dslot = (d0, d1)
        gsem = (sgs0, sgs1)
        dsem = (sgd0, sgd1)

        cpi = pltpu.make_async_copy(
            eix_hbm.at[:, pl.ds(wid * cpw, cpw), :], idxb, sgs0)
        cpi.start()

        pltpu.sync_copy(z_hbm.at[pl.ds(s * rpt, rpt)],
                        acc.at[pl.ds(s * rpt, rpt)])

        @pl.when(s == 0)
        def _():
            pltpu.sync_copy(z_hbm.at[pl.ds(rpt * _NS, rem)],
                            acc.at[pl.ds(rpt * _NS, rem)])

        cpi.wait()
        plsc.subcore_barrier()

        def gather_desc(chunk, slot):
            cg = pltpu.make_async_copy(
                tsrc_hbm.at[idxb.at[0, chunk]], gslot[slot], gsem[slot])
            cd = pltpu.make_async_copy(
                tdst_hbm.at[idxb.at[1, chunk]], dslot[slot], dsem[slot])
            return cg, cd

        def issue_gather(chunk, slot):
            cg, cd = gather_desc(chunk, slot)
            cg.start()
            cd.start()

        def wait_gather(chunk, slot):
            cg, cd = gather_desc(chunk, slot)
            cg.wait()
            cd.wait()

        def compute(slot):
            g, d = gslot[slot], dslot[slot]

            @plsc.parallel_loop(0, _C, 1, unroll=4)
            def _(ei):
                a = g[ei, pl.ds(128, _L)] + d[ei, pl.ds(0, _L)]
                a = jnp.maximum(a, a * 0.2)
                ex = jnp.exp(a)
                ob[ei, pl.ds(128, _L)] = ex
                for h in range(8):
                    bh = _bcast_lane(ex, h)
                    ob[ei, pl.ds(h * _L, _L)] = g[ei, pl.ds(h * _L, _L)] * bh

        def scatter_desc(chunk):
            return pltpu.make_async_copy(
                ob, acc.at[idxb.at[1, chunk]], ssc)

        issue_gather(0, 0)
        issue_gather(1, 1)

        wait_gather(0, 0)
        compute(0)
        scatter_desc(0).start(add=True)
        issue_gather(2, 0)

        @pl.loop(1, cpw)
        def _(chunk):
            slot = lax.rem(chunk, 2)

            @pl.when(slot == 0)
            def _():
                wait_gather(chunk, 0)

            @pl.when(slot == 1)
            def _():
                wait_gather(chunk, 1)

            scatter_desc(chunk - 1).wait()

            @pl.when(slot == 0)
            def _():
                compute(0)

            @pl.when(slot == 1)
            def _():
                compute(1)

            scatter_desc(chunk).start(add=True)

            @pl.when((slot == 0) & (chunk + 2 < cpw))
            def _():
                issue_gather(chunk + 2, 0)

            @pl.when((slot == 1) & (chunk + 2 < cpw))
            def _():
                issue_gather(chunk + 2, 1)

        scatter_desc(cpw - 1).wait()

        plsc.subcore_barrier()
        pltpu.sync_copy(acc.at[pl.ds(s * rpt, rpt)],
                        out_hbm.at[c, pl.ds(s * rpt, rpt)])

        @pl.when(s == 0)
        def _():
            pltpu.sync_copy(acc.at[pl.ds(rpt * _NS, rem)],
                            out_hbm.at[c, pl.ds(rpt * _NS, rem)])

    return k(tsrc, tdst, eix, zeros_hbm)


def _prep_tc(x, w_ext, w_dst):
    n = x.shape[0]

    def body(x_ref, we_ref, wd_ref, ts_ref, td_ref):
        xx = x_ref[...]
        ts_ref[...] = jnp.dot(xx, we_ref[...], preferred_element_type=F32)
        td_ref[...] = jnp.dot(xx, wd_ref[...], preferred_element_type=F32)

    return pl.pallas_call(
        body,
        out_shape=(jax.ShapeDtypeStruct((n, 144), F32),
                   jax.ShapeDtypeStruct((n, 16), F32)),
    )(x, w_ext, w_dst)


def _mid_tc(acc, b1, w_ext, w_dst, p8):
    n = acc.shape[1]

    def body(acc_ref, b1_ref, we_ref, wd_ref, p8_ref, ts_ref, td_ref):
        a = acc_ref[0] + acc_ref[1]
        num = a[:, :128]
        den = a[:, 128:136]
        r = 1.0 / (den + 1e-16)
        rex = jnp.dot(r, p8_ref[...], preferred_element_type=F32)
        hpre = num * rex + b1_ref[...]
        hh = jnp.where(hpre > 0, hpre, jnp.exp(hpre) - 1.0)
        ts_ref[...] = jnp.dot(hh, we_ref[...], preferred_element_type=F32)
        td_ref[...] = jnp.dot(hh, wd_ref[...], preferred_element_type=F32)

    return pl.pallas_call(
        body,
        out_shape=(jax.ShapeDtypeStruct((n, 144), F32),
                   jax.ShapeDtypeStruct((n, 16), F32)),
    )(acc, b1, w_ext, w_dst, p8)


def _final_tc(acc, b2, p0):
    n = acc.shape[1]

    def body(acc_ref, b2_ref, p0_ref, o_ref):
        a = acc_ref[0] + acc_ref[1]
        num = a[:, :128]
        den = a[:, 128:136]
        r = 1.0 / (den + 1e-16)
        rex = jnp.dot(r, p0_ref[...], preferred_element_type=F32)
        o = num * rex + b2_ref[...]
        m = jnp.max(o, axis=1, keepdims=True)
        z = o - m
        lse = jnp.log(jnp.sum(jnp.exp(z), axis=1, keepdims=True))
        o_ref[...] = z - lse

    return pl.pallas_call(
        body,
        out_shape=jax.ShapeDtypeStruct((n, 128), F32),
    )(acc, b2, p0)


def kernel(x, edge_index, W1, att_src1, att_dst1, b1, W2, att_src2,
           att_dst2, b2):
    n = x.shape[0]
    heads, hid = att_src1.shape
    eix = edge_index.reshape(2, -1, _C)

    eye = jnp.eye(heads, dtype=F32)
    a_s = (eye[:, None, :] * att_src1[:, :, None]).reshape(heads * hid, heads)
    a_d = (eye[:, None, :] * att_dst1[:, :, None]).reshape(heads * hid, heads)
    z8 = jnp.zeros((heads * hid, 8), F32)
    w1_ext = jnp.concatenate([W1, W1 @ a_s, z8], axis=1)
    w1_dst = jnp.concatenate([W1 @ a_d, z8], axis=1)
    w2_ext = jnp.concatenate(
        [W2, jnp.tile((W2 @ att_src2[0])[:, None], (1, 16))], axis=1)
    w2_dst = jnp.tile((W2 @ att_dst2[0])[:, None], (1, 16))
    p8 = (jnp.arange(128)[None, :] // hid
          == jnp.arange(heads)[:, None]).astype(F32)
    p0 = jnp.concatenate([jnp.ones((1, 128), F32),
                          jnp.zeros((7, 128), F32)], axis=0)
    zeros_hbm = jnp.zeros((n, 144), F32)

    tsrc1, tdst1 = _prep_tc(x, w1_ext, w1_dst)
    acc1 = _edge_pass(tsrc1, tdst1, eix, zeros_hbm)
    tsrc2, tdst2 = _mid_tc(acc1, b1.reshape(1, -1), w2_ext, w2_dst, p8)
    acc2 = _edge_pass(tsrc2, tdst2, eix, zeros_hbm)
    return _final_tc(acc2, b2.reshape(1, -1), p0)
